# baseline (speedup 1.0000x reference)
import functools

import jax
import jax.numpy as jnp
from jax import lax
from jax.experimental import pallas as pl
from jax.experimental.pallas import tpu as pltpu
from jax.experimental.pallas import tpu_sc as plsc

NUM_NODES = 100000
WINDOW = 6
DIM = 128
BATCH = 16384

SPLIT = 8192

NUM_CORES = 2
NUM_SUBCORES = 16
NUM_WORKERS = NUM_CORES * NUM_SUBCORES
B_PER_W = SPLIT // NUM_WORKERS
CHUNK = 32
NCHUNK = B_PER_W // CHUNK
NBUF = 3


def _sc_gather_body(bank_hbm, idx_hbm, out_hbm, idx_v, *rest):
    bufs = rest[:NBUF]
    gsems = rest[NBUF:2 * NBUF]
    ssems = rest[2 * NBUF:3 * NBUF]

    wid = lax.axis_index("s") * NUM_CORES + lax.axis_index("c")
    base = wid * B_PER_W
    pltpu.sync_copy(idx_hbm.at[pl.ds(base, B_PER_W)], idx_v)

    gathers = [None] * NBUF
    scatters = [None] * NBUF

    def start_gather(c):
        b = c % NBUF
        if scatters[b] is not None:
            scatters[b].wait()
        gathers[b] = pltpu.async_copy(
            bank_hbm.at[idx_v.at[pl.ds(c * CHUNK, CHUNK)]], bufs[b], gsems[b])

    def finish_chunk(c):
        b = c % NBUF
        gathers[b].wait()
        scatters[b] = pltpu.async_copy(
            bufs[b], out_hbm.at[pl.ds(base + c * CHUNK, CHUNK)], ssems[b])

    depth = NBUF - 1
    for c in range(NCHUNK):
        start_gather(c)
        if c >= depth:
            finish_chunk(c - depth)
    for c in range(NCHUNK - depth, NCHUNK):
        finish_chunk(c)
    for b in range(NBUF):
        if scatters[b] is not None:
            scatters[b].wait()


def _sc_gather(bank, idx):
    mesh = plsc.VectorSubcoreMesh(core_axis_name="c", subcore_axis_name="s")
    k = functools.partial(
        pl.kernel,
        out_type=jax.ShapeDtypeStruct((SPLIT, WINDOW, DIM), jnp.float32),
        mesh=mesh,
        scratch_types=(
            [pltpu.VMEM((B_PER_W,), jnp.int32)]
            + [pltpu.VMEM((CHUNK, WINDOW, DIM), jnp.float32)
               for _ in range(NBUF)]
            + [pltpu.SemaphoreType.DMA for _ in range(2 * NBUF)]
        ),
    )(_sc_gather_body)
    return k(bank, idx)


TC_ROWS = BATCH - SPLIT
TC_GROUP = 64
TC_NGROUP = TC_ROWS // TC_GROUP


def _tc_gather_body(idx_smem, bank_any, out_any, vbuf, sem_in, sem_out):
    def group(g, carry):
        sel = lax.rem(g, 2)

        @pl.when(g >= 2)
        def _():
            pltpu.make_async_copy(
                vbuf.at[pl.ds(sel * TC_GROUP, TC_GROUP)],
                out_any.at[pl.ds((g - 2) * TC_GROUP, TC_GROUP)],
                sem_out).wait()

        base = SPLIT + g * TC_GROUP
        copies = []
        for i in range(TC_GROUP):
            row = idx_smem[base + i]
            cp = pltpu.make_async_copy(
                bank_any.at[pl.ds(row, 1)],
                vbuf.at[pl.ds(sel * TC_GROUP + i, 1)],
                sem_in)
            cp.start()
            copies.append(cp)
        for cp in copies:
            cp.wait()
        out_cp = pltpu.make_async_copy(
            vbuf.at[pl.ds(sel * TC_GROUP, TC_GROUP)],
            out_any.at[pl.ds(g * TC_GROUP, TC_GROUP)],
            sem_out)
        out_cp.start()
        return carry

    lax.fori_loop(0, TC_NGROUP, group, 0)
    for k in (TC_NGROUP - 2, TC_NGROUP - 1):
        pltpu.make_async_copy(
            vbuf.at[pl.ds((k % 2) * TC_GROUP, TC_GROUP)],
            out_any.at[pl.ds(k * TC_GROUP, TC_GROUP)],
            sem_out).wait()


def _tc_gather(bank, idx):
    return pl.pallas_call(
        _tc_gather_body,
        in_specs=[
            pl.BlockSpec(memory_space=pltpu.SMEM),
            pl.BlockSpec(memory_space=pl.ANY),
        ],
        out_specs=pl.BlockSpec(memory_space=pl.ANY),
        out_shape=jax.ShapeDtypeStruct((TC_ROWS, WINDOW, DIM), jnp.float32),
        scratch_shapes=[
            pltpu.VMEM((2 * TC_GROUP, WINDOW, DIM), jnp.float32),
            pltpu.SemaphoreType.DMA,
            pltpu.SemaphoreType.DMA,
        ],
    )(idx, bank)


ROWS_BLK = 1024
A_BLKS = SPLIT // ROWS_BLK


def _assemble_body(a_ref, b_ref, evo_ref, cons_ref):
    i = pl.program_id(0)

    @pl.when(i < A_BLKS)
    def _():
        evo_ref[...] = a_ref[...]

    @pl.when(i >= A_BLKS)
    def _():
        evo_ref[...] = b_ref[...]

    x = evo_ref[...]
    n2 = jnp.sum(x * x, axis=-1)
    n = jnp.maximum(jnp.sqrt(n2), 1e-6)
    dot = jnp.sum(x[:, :-1, :] * x[:, 1:, :], axis=-1)
    sim = dot / (n[:, :-1] * n[:, 1:])
    mean = jnp.mean(sim, axis=-1, keepdims=True)
    var = jnp.sum((sim - mean) ** 2, axis=-1) / (WINDOW - 2)
    std = jnp.sqrt(var)
    cons_ref[...] = jnp.clip(1.0 / (1.0 + std), 0.0, 1.0)[:, None]


def _assemble(evo_a, evo_b):
    return pl.pallas_call(
        _assemble_body,
        grid=(BATCH // ROWS_BLK,),
        in_specs=[
            pl.BlockSpec((ROWS_BLK, WINDOW, DIM),
                         lambda i: (jnp.minimum(i, A_BLKS - 1), 0, 0)),
            pl.BlockSpec((ROWS_BLK, WINDOW, DIM),
                         lambda i: (jnp.maximum(i - A_BLKS, 0), 0, 0)),
        ],
        out_specs=[
            pl.BlockSpec((ROWS_BLK, WINDOW, DIM), lambda i: (i, 0, 0)),
            pl.BlockSpec((ROWS_BLK, 1), lambda i: (i, 0)),
        ],
        out_shape=[
            jax.ShapeDtypeStruct((BATCH, WINDOW, DIM), jnp.float32),
            jax.ShapeDtypeStruct((BATCH, 1), jnp.float32),
        ],
    )(evo_a, evo_b)


def kernel(bank, idx):
    evo_a = _sc_gather(bank, idx)
    evo_b = _tc_gather(bank, idx)
    evo, cons = _assemble(evo_a, evo_b)
    return evo, cons.reshape(BATCH)


# speedup vs baseline: 1.2402x; 1.2402x over previous
import functools

import jax
import jax.numpy as jnp
from jax import lax
from jax.experimental import pallas as pl
from jax.experimental.pallas import tpu as pltpu
from jax.experimental.pallas import tpu_sc as plsc

NUM_NODES = 100000
WINDOW = 6
DIM = 128
BATCH = 16384

NUM_CORES = 2
NUM_SUBCORES = 16
NUM_WORKERS = NUM_CORES * NUM_SUBCORES
B_PER_W = BATCH // NUM_WORKERS
CHUNK = 32
NCHUNK = B_PER_W // CHUNK
NBUF = 3


def _sc_gather_body(bank_hbm, idx_hbm, out_hbm, idx_v, *rest):
    bufs = rest[:NBUF]
    gsems = rest[NBUF:2 * NBUF]
    ssems = rest[2 * NBUF:3 * NBUF]

    wid = lax.axis_index("s") * NUM_CORES + lax.axis_index("c")
    base = wid * B_PER_W
    pltpu.sync_copy(idx_hbm.at[pl.ds(base, B_PER_W)], idx_v)

    gathers = [None] * NBUF
    scatters = [None] * NBUF

    def start_gather(c):
        b = c % NBUF
        if scatters[b] is not None:
            scatters[b].wait()
        gathers[b] = pltpu.async_copy(
            bank_hbm.at[idx_v.at[pl.ds(c * CHUNK, CHUNK)]], bufs[b], gsems[b])

    def finish_chunk(c):
        b = c % NBUF
        gathers[b].wait()
        scatters[b] = pltpu.async_copy(
            bufs[b], out_hbm.at[pl.ds(base + c * CHUNK, CHUNK)], ssems[b])

    depth = NBUF - 1
    for c in range(NCHUNK):
        start_gather(c)
        if c >= depth:
            finish_chunk(c - depth)
    for c in range(NCHUNK - depth, NCHUNK):
        finish_chunk(c)
    for b in range(NBUF):
        if scatters[b] is not None:
            scatters[b].wait()


def _sc_gather(bank, idx):
    mesh = plsc.VectorSubcoreMesh(core_axis_name="c", subcore_axis_name="s")
    k = functools.partial(
        pl.kernel,
        out_type=jax.ShapeDtypeStruct((BATCH, WINDOW, DIM), jnp.float32),
        mesh=mesh,
        compiler_params=pltpu.CompilerParams(use_tc_tiling_on_sc=True),
        scratch_types=(
            [pltpu.VMEM((B_PER_W,), jnp.int32)]
            + [pltpu.VMEM((CHUNK, WINDOW, DIM), jnp.float32)
               for _ in range(NBUF)]
            + [pltpu.SemaphoreType.DMA for _ in range(2 * NBUF)]
        ),
    )(_sc_gather_body)
    return k(bank, idx)


ROWS_BLK = 1024


def _consistency_body(evo_ref, out_ref):
    x = evo_ref[...]
    n2 = jnp.sum(x * x, axis=-1)
    n = jnp.maximum(jnp.sqrt(n2), 1e-6)
    dot = jnp.sum(x[:, :-1, :] * x[:, 1:, :], axis=-1)
    sim = dot / (n[:, :-1] * n[:, 1:])
    mean = jnp.mean(sim, axis=-1, keepdims=True)
    var = jnp.sum((sim - mean) ** 2, axis=-1) / (WINDOW - 2)
    std = jnp.sqrt(var)
    out_ref[...] = jnp.clip(1.0 / (1.0 + std), 0.0, 1.0)[:, None]


def _consistency(evo):
    return pl.pallas_call(
        _consistency_body,
        grid=(BATCH // ROWS_BLK,),
        in_specs=[pl.BlockSpec((ROWS_BLK, WINDOW, DIM), lambda i: (i, 0, 0))],
        out_specs=pl.BlockSpec((ROWS_BLK, 1), lambda i: (i, 0)),
        out_shape=jax.ShapeDtypeStruct((BATCH, 1), jnp.float32),
    )(evo)


def kernel(bank, idx):
    evo = _sc_gather(bank, idx)
    cons = _consistency(evo).reshape(BATCH)
    return evo, cons
